# baseline (speedup 1.0000x reference)
import functools

import jax
import jax.numpy as jnp
from jax import lax
from jax.experimental import pallas as pl
from jax.experimental.pallas import tpu as pltpu
from jax.experimental.pallas import tpu_sc as plsc

N = 100000
E = 1600000
G_OUT = 2000
F = 50
NP = 100096
NC = 2
NS = 16
NW = NC * NS
SLICE = NP // NS
EPW = E // NW
CHUNK = 2000
NITER = EPW // CHUNK
PAIRS = (NITER - 1) // 2


def _zero_fill(vref, nwords):
    def body(i, carry):
        vref[pl.ds(i * 16, 16)] = jnp.zeros((16,), jnp.float32)
        return carry
    lax.fori_loop(0, nwords // 16, body, 0)


MAGIC = 0x5F3759DF


def _rsqrt16(d):
    ii = lax.bitcast_convert_type(d, jnp.int32)
    ii = jnp.full((16,), MAGIC, jnp.int32) - lax.shift_right_logical(
        ii, jnp.full((16,), 1, jnp.int32))
    y = lax.bitcast_convert_type(ii, jnp.float32)
    for _ in range(3):
        y = y * (1.5 - 0.5 * d * y * y)
    return y


def _deg_body(edge_hbm, out_hbm, idx_a, idx_b, ones_v, zbuf_v, sem_a, sem_b,
              deg_sh):
    c = lax.axis_index("c")
    s = lax.axis_index("s")
    wid = s * NC + c
    base_w = wid * EPW

    def ofill(i, carry):
        ones_v[pl.ds(i * 16, 16)] = jnp.full((16,), 1.0, jnp.float32)
        return carry
    lax.fori_loop(0, CHUNK // 16, ofill, 0)
    _zero_fill(zbuf_v, SLICE)

    sl = pl.ds(s * SLICE, SLICE)
    pltpu.sync_copy(zbuf_v, deg_sh.at[sl])
    plsc.subcore_barrier()

    def start(k, buf, sem):
        pltpu.async_copy(edge_hbm.at[pl.ds(E + base_w + k * CHUNK, CHUNK)],
                         buf, sem)

    def wait(buf, sem):
        pltpu.make_async_copy(edge_hbm.at[pl.ds(base_w, CHUNK)], buf,
                              sem).wait()

    def scat(buf):
        pltpu.sync_copy(ones_v, deg_sh.at[buf], add=True)

    start(0, idx_a, sem_a)

    def body(j, carry):
        start(2 * j + 1, idx_b, sem_b)
        wait(idx_a, sem_a)
        scat(idx_a)
        start(2 * j + 2, idx_a, sem_a)
        wait(idx_b, sem_b)
        scat(idx_b)
        return carry
    lax.fori_loop(0, PAIRS, body, 0)
    wait(idx_a, sem_a)
    scat(idx_a)

    plsc.subcore_barrier()
    pltpu.sync_copy(deg_sh.at[sl], zbuf_v)
    pltpu.sync_copy(zbuf_v, out_hbm.at[pl.ds(c * NP + s * SLICE, SLICE)])


LAST = N - (NS - 1) * SLICE


def _msg_body(edge_hbm, x_hbm, degp_hbm, w1_hbm,
              u0_out, u1_out, t_out,
              src_a, src_b, dst_a, dst_b, val_a, val_b, xbuf, p0buf, p1buf,
              tbuf, dinvbuf, zbuf_v, wbuf,
              sem_ea, sem_eb, sem_ga, sem_gb,
              acc_sh):
    c = lax.axis_index("c")
    s = lax.axis_index("s")
    wid = s * NC + c
    base_w = wid * EPW
    sl = pl.ds(s * SLICE, SLICE)

    pltpu.sync_copy(x_hbm.at[sl], xbuf)
    pltpu.sync_copy(degp_hbm.at[pl.ds(s * SLICE, SLICE)], p0buf)
    pltpu.sync_copy(degp_hbm.at[pl.ds(NP + s * SLICE, SLICE)], p1buf)
    pltpu.sync_copy(w1_hbm, wbuf)
    w = wbuf[...]

    def node(i, carry):
        ix = pl.ds(i * 16, 16)
        d = p0buf[ix] + p1buf[ix] + 1.0
        y = _rsqrt16(d)
        dinvbuf[ix] = y
        tbuf[ix] = xbuf[ix] * y * w
        return carry
    lax.fori_loop(0, SLICE // 16, node, 0)

    pltpu.sync_copy(tbuf, t_out.at[pl.ds(c * NP + s * SLICE, SLICE)])
    _zero_fill(zbuf_v, SLICE)
    pltpu.sync_copy(zbuf_v, acc_sh.at[sl])
    plsc.subcore_barrier()

    t_core = t_out.at[pl.ds(c * NP, NP)]

    def start_e(k, sbuf, dbuf, esem):
        pltpu.async_copy(edge_hbm.at[pl.ds(base_w + k * CHUNK, CHUNK)],
                         sbuf, esem)
        pltpu.async_copy(edge_hbm.at[pl.ds(E + base_w + k * CHUNK, CHUNK)],
                         dbuf, esem)

    def wait_e(sbuf, dbuf, esem):
        e = pl.ds(base_w, CHUNK)
        pltpu.make_async_copy(edge_hbm.at[e], sbuf, esem).wait()
        pltpu.make_async_copy(edge_hbm.at[e], dbuf, esem).wait()

    def start_g(sbuf, vbuf, gsem):
        pltpu.async_copy(t_core.at[sbuf], vbuf, gsem)

    def wait_g(sbuf, vbuf, gsem):
        pltpu.make_async_copy(t_core.at[sbuf], vbuf, gsem).wait()

    def scat(vbuf, dbuf):
        pltpu.sync_copy(vbuf, acc_sh.at[dbuf], add=True)

    start_e(0, src_a, dst_a, sem_ea)
    wait_e(src_a, dst_a, sem_ea)
    start_g(src_a, val_a, sem_ga)
    start_e(1, src_b, dst_b, sem_eb)

    def body(j, carry):
        wait_e(src_b, dst_b, sem_eb)
        wait_g(src_a, val_a, sem_ga)
        start_g(src_b, val_b, sem_gb)
        scat(val_a, dst_a)
        start_e(2 * j + 2, src_a, dst_a, sem_ea)
        wait_e(src_a, dst_a, sem_ea)
        wait_g(src_b, val_b, sem_gb)
        start_g(src_a, val_a, sem_ga)
        scat(val_b, dst_b)

        @pl.when(j < PAIRS - 1)
        def _():
            start_e(2 * j + 3, src_b, dst_b, sem_eb)
        return carry
    lax.fori_loop(0, PAIRS, body, 0)
    wait_g(src_a, val_a, sem_ga)
    scat(val_a, dst_a)

    plsc.subcore_barrier()
    pltpu.sync_copy(acc_sh.at[sl], p0buf)

    def post(i, carry):
        ix = pl.ds(i * 16, 16)
        p0buf[ix] = dinvbuf[ix] * (p0buf[ix] + 0.5 * tbuf[ix])
        return carry
    lax.fori_loop(0, SLICE // 16, post, 0)

    def wr(out_ref):
        @pl.when(s < NS - 1)
        def _():
            pltpu.sync_copy(p0buf, out_ref.at[sl])

        @pl.when(s == NS - 1)
        def _():
            pltpu.sync_copy(p0buf.at[pl.ds(0, LAST)],
                            out_ref.at[pl.ds((NS - 1) * SLICE, LAST)])

    @pl.when(c == 0)
    def _():
        wr(u0_out)

    @pl.when(c == 1)
    def _():
        wr(u1_out)


def _head_body(u0_ref, u1_ref, wfc_ref, bfc_ref, b1_ref, off_ref, out_ref):
    xg = u0_ref[...] + u1_ref[...] + b1_ref[0] + off_ref[0]
    out_ref[...] = (jnp.dot(xg, wfc_ref[...], preferred_element_type=jnp.float32)
                    + bfc_ref[...])


@functools.lru_cache(maxsize=1)
def _sc_calls():
    mesh = plsc.VectorSubcoreMesh(core_axis_name="c", subcore_axis_name="s",
                                  num_cores=NC, num_subcores=NS)
    deg_call = pl.kernel(
        _deg_body,
        out_type=jax.ShapeDtypeStruct((2 * NP,), jnp.float32),
        mesh=mesh,
        scratch_types=[
            pltpu.VMEM((CHUNK,), jnp.int32),
            pltpu.VMEM((CHUNK,), jnp.int32),
            pltpu.VMEM((CHUNK,), jnp.float32),
            pltpu.VMEM((SLICE,), jnp.float32),
            pltpu.SemaphoreType.DMA,
            pltpu.SemaphoreType.DMA,
            pltpu.VMEM_SHARED((NP,), jnp.float32),
        ],
    )
    msg_call = pl.kernel(
        _msg_body,
        out_type=[jax.ShapeDtypeStruct((N,), jnp.float32),
                  jax.ShapeDtypeStruct((N,), jnp.float32),
                  jax.ShapeDtypeStruct((2 * NP,), jnp.float32)],
        mesh=mesh,
        scratch_types=[
            pltpu.VMEM((CHUNK,), jnp.int32),
            pltpu.VMEM((CHUNK,), jnp.int32),
            pltpu.VMEM((CHUNK,), jnp.int32),
            pltpu.VMEM((CHUNK,), jnp.int32),
            pltpu.VMEM((CHUNK,), jnp.float32),
            pltpu.VMEM((CHUNK,), jnp.float32),
            pltpu.VMEM((SLICE,), jnp.float32),
            pltpu.VMEM((SLICE,), jnp.float32),
            pltpu.VMEM((SLICE,), jnp.float32),
            pltpu.VMEM((SLICE,), jnp.float32),
            pltpu.VMEM((SLICE,), jnp.float32),
            pltpu.VMEM((SLICE,), jnp.float32),
            pltpu.VMEM((16,), jnp.float32),
            pltpu.SemaphoreType.DMA,
            pltpu.SemaphoreType.DMA,
            pltpu.SemaphoreType.DMA,
            pltpu.SemaphoreType.DMA,
            pltpu.VMEM_SHARED((NP,), jnp.float32),
        ],
    )
    return deg_call, msg_call


def kernel(x, edge_index, num_graphs, W1, b1, Wfc, bfc):
    deg_call, msg_call = _sc_calls()
    x_pad = jnp.pad(x.reshape(N), (0, NP - N))
    w1p = jnp.broadcast_to(W1.reshape(1), (16,))

    edge_flat = edge_index.reshape(2 * E)
    deg_p = deg_call(edge_flat)
    u0, u1, _ = msg_call(edge_flat, x_pad, deg_p, w1p)

    off = (jnp.asarray(num_graphs, jnp.float32) - G_OUT).reshape(1)

    logits = pl.pallas_call(
        _head_body,
        out_shape=jax.ShapeDtypeStruct((G_OUT, 2), jnp.float32),
        in_specs=[pl.BlockSpec(memory_space=pltpu.VMEM)] * 4
        + [pl.BlockSpec(memory_space=pltpu.SMEM)] * 2,
    )(u0.reshape(G_OUT, F), u1.reshape(G_OUT, F), Wfc, bfc.reshape(1, 2),
      b1.reshape(1), off)

    reg = jnp.zeros((0,), jnp.float32)
    return (logits, reg)


# speedup vs baseline: 1.2014x; 1.2014x over previous
import functools

import jax
import jax.numpy as jnp
from jax import lax
from jax.experimental import pallas as pl
from jax.experimental.pallas import tpu as pltpu
from jax.experimental.pallas import tpu_sc as plsc

N = 100000
E = 1600000
G_OUT = 2000
F = 50
NP = 100096
NC = 2
NS = 16
NW = NC * NS
SLICE = NP // NS
EPW = E // NW
CHUNK = 2560
NCHUNKS = E // CHUNK
NWMAX = 20
PAIRS = NWMAX // 2


def _zero_fill(vref, nwords):
    def body(i, carry):
        vref[pl.ds(i * 16, 16)] = jnp.zeros((16,), jnp.float32)
        return carry
    lax.fori_loop(0, nwords // 16, body, 0)


MAGIC = 0x5F3759DF


def _rsqrt16(d):
    ii = lax.bitcast_convert_type(d, jnp.int32)
    ii = jnp.full((16,), MAGIC, jnp.int32) - lax.shift_right_logical(
        ii, jnp.full((16,), 1, jnp.int32))
    y = lax.bitcast_convert_type(ii, jnp.float32)
    for _ in range(3):
        y = y * (1.5 - 0.5 * d * y * y)
    return y


def _deg_body(edge_hbm, out_hbm, idx_a, idx_b, dstbuf, ones_v, zbuf_v,
              sem_a, sem_b, deg_sh):
    c = lax.axis_index("c")
    s = lax.axis_index("s")
    wid = s * NC + c
    base_w = wid * EPW

    def ofill(i, carry):
        ones_v[pl.ds(i * 16, 16)] = jnp.full((16,), 1.0, jnp.float32)
        return carry
    lax.fori_loop(0, CHUNK // 16, ofill, 0)
    _zero_fill(zbuf_v, SLICE)

    sl = pl.ds(s * SLICE, SLICE)
    pltpu.sync_copy(zbuf_v, deg_sh.at[sl])
    plsc.subcore_barrier()

    nw = jnp.where(wid < 17, 20, 19)

    def start(j, buf, sem):
        g = (wid + 32 * j) * CHUNK
        pltpu.async_copy(edge_hbm.at[:, pl.ds(g, CHUNK)], buf, sem)

    def wait(buf, sem):
        pltpu.make_async_copy(edge_hbm.at[:, pl.ds(0, CHUNK)], buf,
                              sem).wait()

    def scat(buf):
        def cp(i, carry):
            ix = pl.ds(i * 16, 16)
            dstbuf[ix] = buf[1, ix]
            return carry
        lax.fori_loop(0, CHUNK // 16, cp, 0)
        pltpu.sync_copy(ones_v, deg_sh.at[dstbuf], add=True)

    start(0, idx_a, sem_a)

    def body(p, carry):
        @pl.when(2 * p + 1 < nw)
        def _():
            start(2 * p + 1, idx_b, sem_b)
        wait(idx_a, sem_a)
        scat(idx_a)

        @pl.when(2 * p + 2 < nw)
        def _():
            start(2 * p + 2, idx_a, sem_a)

        @pl.when(2 * p + 1 < nw)
        def _():
            wait(idx_b, sem_b)
            scat(idx_b)
        return carry
    lax.fori_loop(0, PAIRS, body, 0)

    plsc.subcore_barrier()
    pltpu.sync_copy(deg_sh.at[sl], zbuf_v)
    pltpu.sync_copy(zbuf_v, out_hbm.at[pl.ds(c * NP + s * SLICE, SLICE)])


LAST = N - (NS - 1) * SLICE


def _msg_body(edge_hbm, x_hbm, degp_hbm, w1_hbm,
              u0_out, u1_out,
              ebuf_a, ebuf_b, srcbuf, dstbuf, val_v, xbuf, p0buf, p1buf,
              tbuf, dinvbuf, zbuf_v, wbuf,
              sem_ea, sem_eb,
              t_sh, acc_sh):
    c = lax.axis_index("c")
    s = lax.axis_index("s")
    wid = s * NC + c
    base_w = wid * EPW
    sl = pl.ds(s * SLICE, SLICE)

    pltpu.sync_copy(x_hbm.at[sl], xbuf)
    pltpu.sync_copy(degp_hbm.at[pl.ds(s * SLICE, SLICE)], p0buf)
    pltpu.sync_copy(degp_hbm.at[pl.ds(NP + s * SLICE, SLICE)], p1buf)
    pltpu.sync_copy(w1_hbm, wbuf)
    w = wbuf[...]

    def node(i, carry):
        ix = pl.ds(i * 16, 16)
        d = p0buf[ix] + p1buf[ix] + 1.0
        y = _rsqrt16(d)
        dinvbuf[ix] = y
        tbuf[ix] = xbuf[ix] * y * w
        return carry
    lax.fori_loop(0, SLICE // 16, node, 0)

    pltpu.sync_copy(tbuf, t_sh.at[sl])
    _zero_fill(zbuf_v, SLICE)
    pltpu.sync_copy(zbuf_v, acc_sh.at[sl])
    plsc.subcore_barrier()

    nw = jnp.where(wid < 17, 20, 19)

    def start_e(j, buf, esem):
        g = (wid + 32 * j) * CHUNK
        pltpu.async_copy(edge_hbm.at[:, pl.ds(g, CHUNK)], buf, esem)

    def wait_e(buf, esem):
        pltpu.make_async_copy(edge_hbm.at[:, pl.ds(0, CHUNK)], buf,
                              esem).wait()

    def proc(buf):
        def cp(i, carry):
            ix = pl.ds(i * 16, 16)
            srcbuf[ix] = buf[0, ix]
            dstbuf[ix] = buf[1, ix]
            return carry
        lax.fori_loop(0, CHUNK // 16, cp, 0)
        pltpu.sync_copy(t_sh.at[srcbuf], val_v)
        pltpu.sync_copy(val_v, acc_sh.at[dstbuf], add=True)

    start_e(0, ebuf_a, sem_ea)

    def body(p, carry):
        @pl.when(2 * p + 1 < nw)
        def _():
            start_e(2 * p + 1, ebuf_b, sem_eb)
        wait_e(ebuf_a, sem_ea)
        proc(ebuf_a)

        @pl.when(2 * p + 2 < nw)
        def _():
            start_e(2 * p + 2, ebuf_a, sem_ea)

        @pl.when(2 * p + 1 < nw)
        def _():
            wait_e(ebuf_b, sem_eb)
            proc(ebuf_b)
        return carry
    lax.fori_loop(0, PAIRS, body, 0)

    plsc.subcore_barrier()
    pltpu.sync_copy(acc_sh.at[sl], p0buf)

    def post(i, carry):
        ix = pl.ds(i * 16, 16)
        p0buf[ix] = dinvbuf[ix] * (p0buf[ix] + 0.5 * tbuf[ix])
        return carry
    lax.fori_loop(0, SLICE // 16, post, 0)

    def wr(out_ref):
        @pl.when(s < NS - 1)
        def _():
            pltpu.sync_copy(p0buf, out_ref.at[sl])

        @pl.when(s == NS - 1)
        def _():
            pltpu.sync_copy(p0buf.at[pl.ds(0, LAST)],
                            out_ref.at[pl.ds((NS - 1) * SLICE, LAST)])

    @pl.when(c == 0)
    def _():
        wr(u0_out)

    @pl.when(c == 1)
    def _():
        wr(u1_out)


def _head_body(u0_ref, u1_ref, wfc_ref, bfc_ref, b1_ref, off_ref, out_ref):
    xg = u0_ref[...] + u1_ref[...] + b1_ref[0] + off_ref[0]
    out_ref[...] = (jnp.dot(xg, wfc_ref[...], preferred_element_type=jnp.float32)
                    + bfc_ref[...])


@functools.lru_cache(maxsize=1)
def _sc_calls():
    mesh = plsc.VectorSubcoreMesh(core_axis_name="c", subcore_axis_name="s",
                                  num_cores=NC, num_subcores=NS)
    deg_call = pl.kernel(
        _deg_body,
        out_type=jax.ShapeDtypeStruct((2 * NP,), jnp.float32),
        mesh=mesh,
        scratch_types=[
            pltpu.VMEM((2, CHUNK), jnp.int32),
            pltpu.VMEM((2, CHUNK), jnp.int32),
            pltpu.VMEM((CHUNK,), jnp.int32),
            pltpu.VMEM((CHUNK,), jnp.float32),
            pltpu.VMEM((SLICE,), jnp.float32),
            pltpu.SemaphoreType.DMA,
            pltpu.SemaphoreType.DMA,
            pltpu.VMEM_SHARED((NP,), jnp.float32),
        ],
    )
    msg_call = pl.kernel(
        _msg_body,
        out_type=[jax.ShapeDtypeStruct((N,), jnp.float32),
                  jax.ShapeDtypeStruct((N,), jnp.float32)],
        mesh=mesh,
        scratch_types=[
            pltpu.VMEM((2, CHUNK), jnp.int32),
            pltpu.VMEM((2, CHUNK), jnp.int32),
            pltpu.VMEM((CHUNK,), jnp.int32),
            pltpu.VMEM((CHUNK,), jnp.int32),
            pltpu.VMEM((CHUNK,), jnp.float32),
            pltpu.VMEM((SLICE,), jnp.float32),
            pltpu.VMEM((SLICE,), jnp.float32),
            pltpu.VMEM((SLICE,), jnp.float32),
            pltpu.VMEM((SLICE,), jnp.float32),
            pltpu.VMEM((SLICE,), jnp.float32),
            pltpu.VMEM((SLICE,), jnp.float32),
            pltpu.VMEM((16,), jnp.float32),
            pltpu.SemaphoreType.DMA,
            pltpu.SemaphoreType.DMA,
            pltpu.VMEM_SHARED((NP,), jnp.float32),
            pltpu.VMEM_SHARED((NP,), jnp.float32),
        ],
    )
    return deg_call, msg_call


def kernel(x, edge_index, num_graphs, W1, b1, Wfc, bfc):
    deg_call, msg_call = _sc_calls()
    x_pad = jnp.pad(x.reshape(N), (0, NP - N))
    w1p = jnp.broadcast_to(W1.reshape(1), (16,))

    deg_p = deg_call(edge_index)
    u0, u1 = msg_call(edge_index, x_pad, deg_p, w1p)

    off = (jnp.asarray(num_graphs, jnp.float32) - G_OUT).reshape(1)

    logits = pl.pallas_call(
        _head_body,
        out_shape=jax.ShapeDtypeStruct((G_OUT, 2), jnp.float32),
        in_specs=[pl.BlockSpec(memory_space=pltpu.VMEM)] * 4
        + [pl.BlockSpec(memory_space=pltpu.SMEM)] * 2,
    )(u0.reshape(G_OUT, F), u1.reshape(G_OUT, F), Wfc, bfc.reshape(1, 2),
      b1.reshape(1), off)

    reg = jnp.zeros((0,), jnp.float32)
    return (logits, reg)


# speedup vs baseline: 1.2196x; 1.0151x over previous
import functools

import jax
import jax.numpy as jnp
from jax import lax
from jax.experimental import pallas as pl
from jax.experimental.pallas import tpu as pltpu
from jax.experimental.pallas import tpu_sc as plsc

N = 100000
E = 1600000
G_OUT = 2000
F = 50
NP = 100096
NC = 2
NS = 16
NW = NC * NS
SLICE = NP // NS
EPW = E // NW
CHUNK = 2560
NCHUNKS = E // CHUNK
NWMAX = 20
PAIRS = NWMAX // 2
ECHUNK = 2000
ENITER = EPW // ECHUNK
EPAIRS = (ENITER - 1) // 2


def _zero_fill(vref, nwords):
    def body(i, carry):
        vref[pl.ds(i * 16, 16)] = jnp.zeros((16,), jnp.float32)
        return carry
    lax.fori_loop(0, nwords // 16, body, 0)


MAGIC = 0x5F3759DF


def _rsqrt16(d):
    ii = lax.bitcast_convert_type(d, jnp.int32)
    ii = jnp.full((16,), MAGIC, jnp.int32) - lax.shift_right_logical(
        ii, jnp.full((16,), 1, jnp.int32))
    y = lax.bitcast_convert_type(ii, jnp.float32)
    for _ in range(3):
        y = y * (1.5 - 0.5 * d * y * y)
    return y


def _deg_body(edge_hbm, out_hbm, idx_a, idx_b, dstbuf, ones_v, zbuf_v,
              sem_a, sem_b, deg_sh):
    c = lax.axis_index("c")
    s = lax.axis_index("s")
    wid = s * NC + c
    base_w = wid * EPW

    def ofill(i, carry):
        ones_v[pl.ds(i * 16, 16)] = jnp.full((16,), 1.0, jnp.float32)
        return carry
    lax.fori_loop(0, CHUNK // 16, ofill, 0)
    _zero_fill(zbuf_v, SLICE)

    sl = pl.ds(s * SLICE, SLICE)
    pltpu.sync_copy(zbuf_v, deg_sh.at[sl])
    plsc.subcore_barrier()

    nw = jnp.where(wid < 17, 20, 19)

    def start(j, buf, sem):
        g = (wid + 32 * j) * CHUNK
        pltpu.async_copy(edge_hbm.at[:, pl.ds(g, CHUNK)], buf, sem)

    def wait(buf, sem):
        pltpu.make_async_copy(edge_hbm.at[:, pl.ds(0, CHUNK)], buf,
                              sem).wait()

    def scat(buf):
        def cp(i, carry):
            ix = pl.ds(i * 16, 16)
            dstbuf[ix] = buf[1, ix]
            return carry
        lax.fori_loop(0, CHUNK // 16, cp, 0)
        pltpu.sync_copy(ones_v, deg_sh.at[dstbuf], add=True)

    start(0, idx_a, sem_a)

    def body(p, carry):
        @pl.when(2 * p + 1 < nw)
        def _():
            start(2 * p + 1, idx_b, sem_b)
        wait(idx_a, sem_a)
        scat(idx_a)

        @pl.when(2 * p + 2 < nw)
        def _():
            start(2 * p + 2, idx_a, sem_a)

        @pl.when(2 * p + 1 < nw)
        def _():
            wait(idx_b, sem_b)
            scat(idx_b)
        return carry
    lax.fori_loop(0, PAIRS, body, 0)

    plsc.subcore_barrier()
    pltpu.sync_copy(deg_sh.at[sl], zbuf_v)
    pltpu.sync_copy(zbuf_v, out_hbm.at[pl.ds(c * NP + s * SLICE, SLICE)])


LAST = N - (NS - 1) * SLICE


def _msg_body(edge_hbm, x_hbm, degp_hbm, w1_hbm,
              u0_out, u1_out,
              srcbuf, srcbuf_b, dstbuf, dstbuf_b, val_v, xbuf, p0buf, p1buf,
              tbuf, dinvbuf, zbuf_v, wbuf,
              sem_ea, sem_eb,
              t_sh, acc_sh):
    c = lax.axis_index("c")
    s = lax.axis_index("s")
    wid = s * NC + c
    base_w = wid * EPW
    sl = pl.ds(s * SLICE, SLICE)

    pltpu.sync_copy(x_hbm.at[sl], xbuf)
    pltpu.sync_copy(degp_hbm.at[pl.ds(s * SLICE, SLICE)], p0buf)
    pltpu.sync_copy(degp_hbm.at[pl.ds(NP + s * SLICE, SLICE)], p1buf)
    pltpu.sync_copy(w1_hbm, wbuf)
    w = wbuf[...]

    def node(i, carry):
        ix = pl.ds(i * 16, 16)
        d = p0buf[ix] + p1buf[ix] + 1.0
        y = _rsqrt16(d)
        dinvbuf[ix] = y
        tbuf[ix] = xbuf[ix] * y * w
        return carry
    lax.fori_loop(0, SLICE // 16, node, 0)

    pltpu.sync_copy(tbuf, t_sh.at[sl])
    _zero_fill(zbuf_v, SLICE)
    pltpu.sync_copy(zbuf_v, acc_sh.at[sl])
    plsc.subcore_barrier()

    def start_e(k, sbuf, dbuf, esem):
        pltpu.async_copy(edge_hbm.at[pl.ds(base_w + k * ECHUNK, ECHUNK)],
                         sbuf, esem)
        pltpu.async_copy(edge_hbm.at[pl.ds(E + base_w + k * ECHUNK, ECHUNK)],
                         dbuf, esem)

    def wait_e(sbuf, dbuf, esem):
        e = pl.ds(base_w, ECHUNK)
        pltpu.make_async_copy(edge_hbm.at[e], sbuf, esem).wait()
        pltpu.make_async_copy(edge_hbm.at[e], dbuf, esem).wait()

    def proc(sbuf, dbuf):
        pltpu.sync_copy(t_sh.at[sbuf], val_v)
        pltpu.sync_copy(val_v, acc_sh.at[dbuf], add=True)

    start_e(0, srcbuf, dstbuf, sem_ea)

    def body(j, carry):
        start_e(2 * j + 1, srcbuf_b, dstbuf_b, sem_eb)
        wait_e(srcbuf, dstbuf, sem_ea)
        proc(srcbuf, dstbuf)
        start_e(2 * j + 2, srcbuf, dstbuf, sem_ea)
        wait_e(srcbuf_b, dstbuf_b, sem_eb)
        proc(srcbuf_b, dstbuf_b)
        return carry
    lax.fori_loop(0, EPAIRS, body, 0)
    wait_e(srcbuf, dstbuf, sem_ea)
    proc(srcbuf, dstbuf)

    plsc.subcore_barrier()
    pltpu.sync_copy(acc_sh.at[sl], p0buf)

    def post(i, carry):
        ix = pl.ds(i * 16, 16)
        p0buf[ix] = dinvbuf[ix] * (p0buf[ix] + 0.5 * tbuf[ix])
        return carry
    lax.fori_loop(0, SLICE // 16, post, 0)

    def wr(out_ref):
        @pl.when(s < NS - 1)
        def _():
            pltpu.sync_copy(p0buf, out_ref.at[sl])

        @pl.when(s == NS - 1)
        def _():
            pltpu.sync_copy(p0buf.at[pl.ds(0, LAST)],
                            out_ref.at[pl.ds((NS - 1) * SLICE, LAST)])

    @pl.when(c == 0)
    def _():
        wr(u0_out)

    @pl.when(c == 1)
    def _():
        wr(u1_out)


def _head_body(u0_ref, u1_ref, wfc_ref, bfc_ref, b1_ref, off_ref, out_ref):
    xg = u0_ref[...] + u1_ref[...] + b1_ref[0] + off_ref[0]
    out_ref[...] = (jnp.dot(xg, wfc_ref[...], preferred_element_type=jnp.float32)
                    + bfc_ref[...])


@functools.lru_cache(maxsize=1)
def _sc_calls():
    mesh = plsc.VectorSubcoreMesh(core_axis_name="c", subcore_axis_name="s",
                                  num_cores=NC, num_subcores=NS)
    deg_call = pl.kernel(
        _deg_body,
        out_type=jax.ShapeDtypeStruct((2 * NP,), jnp.float32),
        mesh=mesh,
        scratch_types=[
            pltpu.VMEM((2, CHUNK), jnp.int32),
            pltpu.VMEM((2, CHUNK), jnp.int32),
            pltpu.VMEM((CHUNK,), jnp.int32),
            pltpu.VMEM((CHUNK,), jnp.float32),
            pltpu.VMEM((SLICE,), jnp.float32),
            pltpu.SemaphoreType.DMA,
            pltpu.SemaphoreType.DMA,
            pltpu.VMEM_SHARED((NP,), jnp.float32),
        ],
    )
    msg_call = pl.kernel(
        _msg_body,
        out_type=[jax.ShapeDtypeStruct((N,), jnp.float32),
                  jax.ShapeDtypeStruct((N,), jnp.float32)],
        mesh=mesh,
        scratch_types=[
            pltpu.VMEM((ECHUNK,), jnp.int32),
            pltpu.VMEM((ECHUNK,), jnp.int32),
            pltpu.VMEM((ECHUNK,), jnp.int32),
            pltpu.VMEM((ECHUNK,), jnp.int32),
            pltpu.VMEM((ECHUNK,), jnp.float32),
            pltpu.VMEM((SLICE,), jnp.float32),
            pltpu.VMEM((SLICE,), jnp.float32),
            pltpu.VMEM((SLICE,), jnp.float32),
            pltpu.VMEM((SLICE,), jnp.float32),
            pltpu.VMEM((SLICE,), jnp.float32),
            pltpu.VMEM((SLICE,), jnp.float32),
            pltpu.VMEM((16,), jnp.float32),
            pltpu.SemaphoreType.DMA,
            pltpu.SemaphoreType.DMA,
            pltpu.VMEM_SHARED((NP,), jnp.float32),
            pltpu.VMEM_SHARED((NP,), jnp.float32),
        ],
    )
    return deg_call, msg_call


def kernel(x, edge_index, num_graphs, W1, b1, Wfc, bfc):
    deg_call, msg_call = _sc_calls()
    x_pad = jnp.pad(x.reshape(N), (0, NP - N))
    w1p = jnp.broadcast_to(W1.reshape(1), (16,))

    edge_flat = edge_index.reshape(2 * E)
    deg_p = deg_call(edge_index)
    u0, u1 = msg_call(edge_flat, x_pad, deg_p, w1p)

    off = (jnp.asarray(num_graphs, jnp.float32) - G_OUT).reshape(1)

    logits = pl.pallas_call(
        _head_body,
        out_shape=jax.ShapeDtypeStruct((G_OUT, 2), jnp.float32),
        in_specs=[pl.BlockSpec(memory_space=pltpu.VMEM)] * 4
        + [pl.BlockSpec(memory_space=pltpu.SMEM)] * 2,
    )(u0.reshape(G_OUT, F), u1.reshape(G_OUT, F), Wfc, bfc.reshape(1, 2),
      b1.reshape(1), off)

    reg = jnp.zeros((0,), jnp.float32)
    return (logits, reg)


# speedup vs baseline: 1.5957x; 1.3084x over previous
import functools

import jax
import jax.numpy as jnp
from jax import lax
from jax.experimental import pallas as pl
from jax.experimental.pallas import tpu as pltpu
from jax.experimental.pallas import tpu_sc as plsc

N = 100000
E = 1600000
G_OUT = 2000
F = 50
NP = 100096
NC = 2
NS = 16
NW = NC * NS
SLICE = NP // NS
EPW = E // NW
CHUNK = 6400
NCHUNKS = E // CHUNK
NWCUT = 26
NWMAX = 8
PAIRS = NWMAX // 2
ECHUNK = 10000
ENITER = EPW // ECHUNK
EPAIRS = (ENITER - 1) // 2


def _zero_fill(vref, nwords):
    def body(i, carry):
        vref[pl.ds(i * 16, 16)] = jnp.zeros((16,), jnp.float32)
        return carry
    lax.fori_loop(0, nwords // 16, body, 0)


MAGIC = 0x5F3759DF


def _rsqrt16(d):
    ii = lax.bitcast_convert_type(d, jnp.int32)
    ii = jnp.full((16,), MAGIC, jnp.int32) - lax.shift_right_logical(
        ii, jnp.full((16,), 1, jnp.int32))
    y = lax.bitcast_convert_type(ii, jnp.float32)
    for _ in range(3):
        y = y * (1.5 - 0.5 * d * y * y)
    return y


def _deg_body(edge_hbm, out_hbm, idx_a, idx_b, dst_a, dst_b, ones_v, zbuf_v,
              sem_a, sem_b, ssem_a, ssem_b, deg_sh):
    c = lax.axis_index("c")
    s = lax.axis_index("s")
    wid = s * NC + c
    base_w = wid * EPW

    def ofill(i, carry):
        ones_v[pl.ds(i * 16, 16)] = jnp.full((16,), 1.0, jnp.float32)
        return carry
    lax.fori_loop(0, CHUNK // 16, ofill, 0)
    _zero_fill(zbuf_v, SLICE)

    sl = pl.ds(s * SLICE, SLICE)
    pltpu.sync_copy(zbuf_v, deg_sh.at[sl])
    plsc.subcore_barrier()

    nw = jnp.where(wid < NWCUT, NWMAX, NWMAX - 1)

    def start(j, buf, sem):
        g = (wid + 32 * j) * CHUNK
        pltpu.async_copy(edge_hbm.at[:, pl.ds(g, CHUNK)], buf, sem)

    def wait(buf, sem):
        pltpu.make_async_copy(edge_hbm.at[:, pl.ds(0, CHUNK)], buf,
                              sem).wait()

    def extract(buf, dbuf):
        def cp(i, carry):
            ix = pl.ds(i * 16, 16)
            dbuf[ix] = buf[1, ix]
            return carry
        lax.fori_loop(0, CHUNK // 16, cp, 0)

    def start_s(dbuf, ssem):
        pltpu.async_copy(ones_v, deg_sh.at[dbuf], ssem, add=True)

    def wait_s(dbuf, ssem):
        pltpu.make_async_copy(ones_v, deg_sh.at[dbuf], ssem).wait()

    start(0, idx_a, sem_a)

    def body(p, carry):
        @pl.when(2 * p + 1 < nw)
        def _():
            start(2 * p + 1, idx_b, sem_b)
        wait(idx_a, sem_a)

        @pl.when(p > 0)
        def _():
            wait_s(dst_a, ssem_a)
        extract(idx_a, dst_a)
        start_s(dst_a, ssem_a)

        @pl.when(2 * p + 2 < nw)
        def _():
            start(2 * p + 2, idx_a, sem_a)

        @pl.when(2 * p + 1 < nw)
        def _():
            wait(idx_b, sem_b)

            @pl.when(p > 0)
            def _():
                wait_s(dst_b, ssem_b)
            extract(idx_b, dst_b)
            start_s(dst_b, ssem_b)
        return carry
    lax.fori_loop(0, PAIRS, body, 0)
    wait_s(dst_a, ssem_a)
    wait_s(dst_b, ssem_b)

    plsc.subcore_barrier()
    pltpu.sync_copy(deg_sh.at[sl], zbuf_v)
    pltpu.sync_copy(zbuf_v, out_hbm.at[pl.ds(c * NP + s * SLICE, SLICE)])


LAST = N - (NS - 1) * SLICE


def _msg_body(edge_hbm, x_hbm, degp_hbm, w1_hbm,
              u0_out, u1_out,
              srcbuf, srcbuf_b, dstbuf, dstbuf_b, val_v, xbuf, p0buf, p1buf,
              tbuf, dinvbuf, zbuf_v, wbuf,
              sem_ea, sem_eb,
              t_sh, acc_sh):
    c = lax.axis_index("c")
    s = lax.axis_index("s")
    wid = s * NC + c
    base_w = wid * EPW
    sl = pl.ds(s * SLICE, SLICE)

    pltpu.async_copy(x_hbm.at[sl], xbuf, sem_ea)
    pltpu.async_copy(degp_hbm.at[pl.ds(s * SLICE, SLICE)], p0buf, sem_ea)
    pltpu.async_copy(degp_hbm.at[pl.ds(NP + s * SLICE, SLICE)], p1buf, sem_ea)
    pltpu.async_copy(w1_hbm, wbuf, sem_ea)
    pltpu.make_async_copy(x_hbm.at[sl], xbuf, sem_ea).wait()
    pltpu.make_async_copy(degp_hbm.at[pl.ds(s * SLICE, SLICE)], p0buf,
                          sem_ea).wait()
    pltpu.make_async_copy(degp_hbm.at[pl.ds(NP + s * SLICE, SLICE)], p1buf,
                          sem_ea).wait()
    pltpu.make_async_copy(w1_hbm, wbuf, sem_ea).wait()
    w = wbuf[...]

    def node(i, carry):
        ix = pl.ds(i * 16, 16)
        d = p0buf[ix] + p1buf[ix] + 1.0
        y = _rsqrt16(d)
        dinvbuf[ix] = y
        tbuf[ix] = xbuf[ix] * y * w
        return carry
    lax.fori_loop(0, SLICE // 16, node, 0)

    pltpu.sync_copy(tbuf, t_sh.at[sl])
    _zero_fill(zbuf_v, SLICE)
    pltpu.sync_copy(zbuf_v, acc_sh.at[sl])
    plsc.subcore_barrier()

    def start_e(k, sbuf, dbuf, esem):
        pltpu.async_copy(edge_hbm.at[pl.ds(base_w + k * ECHUNK, ECHUNK)],
                         sbuf, esem)
        pltpu.async_copy(edge_hbm.at[pl.ds(E + base_w + k * ECHUNK, ECHUNK)],
                         dbuf, esem)

    def wait_e(sbuf, dbuf, esem):
        e = pl.ds(base_w, ECHUNK)
        pltpu.make_async_copy(edge_hbm.at[e], sbuf, esem).wait()
        pltpu.make_async_copy(edge_hbm.at[e], dbuf, esem).wait()

    def proc(sbuf, dbuf):
        pltpu.sync_copy(t_sh.at[sbuf], val_v)
        pltpu.sync_copy(val_v, acc_sh.at[dbuf], add=True)

    start_e(0, srcbuf, dstbuf, sem_ea)

    def body(j, carry):
        start_e(2 * j + 1, srcbuf_b, dstbuf_b, sem_eb)
        wait_e(srcbuf, dstbuf, sem_ea)
        proc(srcbuf, dstbuf)
        start_e(2 * j + 2, srcbuf, dstbuf, sem_ea)
        wait_e(srcbuf_b, dstbuf_b, sem_eb)
        proc(srcbuf_b, dstbuf_b)
        return carry
    lax.fori_loop(0, EPAIRS, body, 0)
    wait_e(srcbuf, dstbuf, sem_ea)
    proc(srcbuf, dstbuf)

    plsc.subcore_barrier()
    pltpu.sync_copy(acc_sh.at[sl], p0buf)

    def post(i, carry):
        ix = pl.ds(i * 16, 16)
        p0buf[ix] = dinvbuf[ix] * (p0buf[ix] + 0.5 * tbuf[ix])
        return carry
    lax.fori_loop(0, SLICE // 16, post, 0)

    def wr(out_ref):
        @pl.when(s < NS - 1)
        def _():
            pltpu.sync_copy(p0buf, out_ref.at[sl])

        @pl.when(s == NS - 1)
        def _():
            pltpu.sync_copy(p0buf.at[pl.ds(0, LAST)],
                            out_ref.at[pl.ds((NS - 1) * SLICE, LAST)])

    @pl.when(c == 0)
    def _():
        wr(u0_out)

    @pl.when(c == 1)
    def _():
        wr(u1_out)


def _head_body(u0_ref, u1_ref, wfc_ref, bfc_ref, b1_ref, off_ref, out_ref):
    xg = u0_ref[...] + u1_ref[...] + b1_ref[0] + off_ref[0]
    out_ref[...] = (jnp.dot(xg, wfc_ref[...], preferred_element_type=jnp.float32)
                    + bfc_ref[...])


@functools.lru_cache(maxsize=1)
def _sc_calls():
    mesh = plsc.VectorSubcoreMesh(core_axis_name="c", subcore_axis_name="s",
                                  num_cores=NC, num_subcores=NS)
    deg_call = pl.kernel(
        _deg_body,
        out_type=jax.ShapeDtypeStruct((2 * NP,), jnp.float32),
        mesh=mesh,
        scratch_types=[
            pltpu.VMEM((2, CHUNK), jnp.int32),
            pltpu.VMEM((2, CHUNK), jnp.int32),
            pltpu.VMEM((CHUNK,), jnp.int32),
            pltpu.VMEM((CHUNK,), jnp.int32),
            pltpu.VMEM((CHUNK,), jnp.float32),
            pltpu.VMEM((SLICE,), jnp.float32),
            pltpu.SemaphoreType.DMA,
            pltpu.SemaphoreType.DMA,
            pltpu.SemaphoreType.DMA,
            pltpu.SemaphoreType.DMA,
            pltpu.VMEM_SHARED((NP,), jnp.float32),
        ],
    )
    msg_call = pl.kernel(
        _msg_body,
        out_type=[jax.ShapeDtypeStruct((N,), jnp.float32),
                  jax.ShapeDtypeStruct((N,), jnp.float32)],
        mesh=mesh,
        scratch_types=[
            pltpu.VMEM((ECHUNK,), jnp.int32),
            pltpu.VMEM((ECHUNK,), jnp.int32),
            pltpu.VMEM((ECHUNK,), jnp.int32),
            pltpu.VMEM((ECHUNK,), jnp.int32),
            pltpu.VMEM((ECHUNK,), jnp.float32),
            pltpu.VMEM((SLICE,), jnp.float32),
            pltpu.VMEM((SLICE,), jnp.float32),
            pltpu.VMEM((SLICE,), jnp.float32),
            pltpu.VMEM((SLICE,), jnp.float32),
            pltpu.VMEM((SLICE,), jnp.float32),
            pltpu.VMEM((SLICE,), jnp.float32),
            pltpu.VMEM((16,), jnp.float32),
            pltpu.SemaphoreType.DMA,
            pltpu.SemaphoreType.DMA,
            pltpu.VMEM_SHARED((NP,), jnp.float32),
            pltpu.VMEM_SHARED((NP,), jnp.float32),
        ],
    )
    return deg_call, msg_call


def kernel(x, edge_index, num_graphs, W1, b1, Wfc, bfc):
    deg_call, msg_call = _sc_calls()
    x_pad = jnp.pad(x.reshape(N), (0, NP - N))
    w1p = jnp.broadcast_to(W1.reshape(1), (16,))

    edge_flat = edge_index.reshape(2 * E)
    deg_p = deg_call(edge_index)
    u0, u1 = msg_call(edge_flat, x_pad, deg_p, w1p)

    off = (jnp.asarray(num_graphs, jnp.float32) - G_OUT).reshape(1)

    logits = pl.pallas_call(
        _head_body,
        out_shape=jax.ShapeDtypeStruct((G_OUT, 2), jnp.float32),
        in_specs=[pl.BlockSpec(memory_space=pltpu.VMEM)] * 4
        + [pl.BlockSpec(memory_space=pltpu.SMEM)] * 2,
    )(u0.reshape(G_OUT, F), u1.reshape(G_OUT, F), Wfc, bfc.reshape(1, 2),
      b1.reshape(1), off)

    reg = jnp.zeros((0,), jnp.float32)
    return (logits, reg)
